# baseline (speedup 1.0000x reference)
import jax
import jax.numpy as jnp
from jax import lax
from jax.experimental import pallas as pl
from jax.experimental.pallas import tpu as pltpu
from jax.experimental.pallas import tpu_sc as plsc

_LQ = 1024
_LK = 1024
_D = 256
_ROWS = 9
_MAXK = 4
_T = 4096
_MID = _T // 2 - 2

_NC = 2
_NS = 16
_ROWS_PER_TILE = _LQ // (_NC * _NS)


def _build_body(off_ref, pe_ref, t8_ref):
    s = pl.program_id(0)
    u = jax.lax.broadcasted_iota(jnp.int32, (_T, _D), 0)
    c = jnp.clip(u + s - _MID + off_ref[0], -_MAXK, _MAXK) + _MAXK
    acc = jnp.zeros((_T, _D), jnp.float32)
    for r in range(_ROWS):
        acc = jnp.where(c == r, pe_ref[r, :][None, :], acc)
    t8_ref[0] = acc


def _build_template(off, pe):
    return pl.pallas_call(
        _build_body,
        grid=(8,),
        in_specs=[
            pl.BlockSpec(memory_space=pltpu.SMEM),
            pl.BlockSpec((_ROWS, _D), lambda s: (0, 0)),
        ],
        out_specs=pl.BlockSpec((1, _T, _D), lambda s: (s, 0, 0)),
        out_shape=jax.ShapeDtypeStruct((8, _T, _D), jnp.float32),
    )(off, pe)


def _sc_copy_body(t8_hbm, out_hbm, sem):
    cid = lax.axis_index("c")
    sid = lax.axis_index("s")
    wid = cid * _NS + sid

    def _row(j, carry):
        q = wid * _ROWS_PER_TILE + j
        start = _MID - q
        s = lax.rem(start, 8)
        base = pl.multiple_of(start - s, 8)
        pltpu.async_copy(
            t8_hbm.at[s, pl.ds(base, _LK), :],
            out_hbm.at[q],
            sem,
        )
        return carry

    lax.fori_loop(0, _ROWS_PER_TILE, _row, 0)

    def _drain(j, carry):
        pltpu.make_async_copy(
            t8_hbm.at[0, pl.ds(0, _LK), :],
            out_hbm.at[0],
            sem,
        ).wait()
        return carry

    lax.fori_loop(0, _ROWS_PER_TILE, _drain, 0)


_sc_copy = pl.kernel(
    _sc_copy_body,
    out_type=jax.ShapeDtypeStruct((_LQ, _LK, _D), jnp.float32),
    mesh=plsc.VectorSubcoreMesh(
        core_axis_name="c", subcore_axis_name="s",
        num_cores=_NC, num_subcores=_NS,
    ),
    scratch_types=[
        pltpu.SemaphoreType.DMA,
    ],
)


def kernel(length_q, length_k, pe):
    off = jnp.asarray(length_k - length_q, jnp.int32).reshape((1,))
    t8 = _build_template(off, pe)
    return _sc_copy(t8)


# speedup vs baseline: 38.4954x; 38.4954x over previous
import jax
import jax.numpy as jnp
from jax import lax
from jax.experimental import pallas as pl
from jax.experimental.pallas import tpu as pltpu
from jax.experimental.pallas import tpu_sc as plsc

_LQ = 1024
_LK = 1024
_D = 256
_ROWS = 9
_MAXK = 4
_T = 4096
_MID = _T // 2 - 2

_NC = 2
_NS = 16
_ROWS_PER_TILE = _LQ // (_NC * _NS)


def _build_body(off_ref, pe_ref, t8_ref):
    s = pl.program_id(0)
    u = jax.lax.broadcasted_iota(jnp.int32, (_T, _D), 0)
    c = jnp.clip(u + s - _MID + off_ref[0], -_MAXK, _MAXK) + _MAXK
    acc = jnp.zeros((_T, _D), jnp.float32)
    for r in range(_ROWS):
        acc = jnp.where(c == r, pe_ref[r, :][None, :], acc)
    t8_ref[0] = acc


def _build_template(off, pe):
    return pl.pallas_call(
        _build_body,
        grid=(8,),
        in_specs=[
            pl.BlockSpec(memory_space=pltpu.SMEM),
            pl.BlockSpec((_ROWS, _D), lambda s: (0, 0)),
        ],
        out_specs=pl.BlockSpec((1, _T, _D), lambda s: (s, 0, 0)),
        out_shape=jax.ShapeDtypeStruct((8, _T, _D), jnp.float32),
    )(off, pe)


_CH = 128
_NCH = _LK // _CH


def _sc_copy_body(t8_hbm, out_hbm, buf_a, buf_b, sem_r, sem_wa, sem_wb):
    cid = lax.axis_index("c")
    sid = lax.axis_index("s")
    wid = cid * _NS + sid

    bufs = (buf_a, buf_b)
    wsems = (sem_wa, sem_wb)

    def _row(j, carry):
        q = wid * _ROWS_PER_TILE + j
        start = _MID - q
        s = lax.rem(start, 8)
        base = pl.multiple_of(start - s, 8)

        def _src(c):
            return t8_hbm.at[s, pl.ds(base + _CH * c, _CH), :]

        def _dst(c):
            return out_hbm.at[q, pl.ds(_CH * c, _CH), :]

        pltpu.async_copy(_src(0), bufs[0], sem_r).wait()
        for c in range(_NCH):
            buf, wsem = bufs[c % 2], wsems[c % 2]
            pltpu.async_copy(buf, _dst(c), wsem)
            if c + 1 < _NCH:
                nbuf, nwsem = bufs[(c + 1) % 2], wsems[(c + 1) % 2]
                if c >= 1:
                    pltpu.make_async_copy(nbuf, _dst(c - 1), nwsem).wait()
                pltpu.async_copy(_src(c + 1), nbuf, sem_r).wait()
        pltpu.make_async_copy(bufs[0], _dst(_NCH - 2), wsems[0]).wait()
        pltpu.make_async_copy(bufs[1], _dst(_NCH - 1), wsems[1]).wait()
        return carry

    lax.fori_loop(0, _ROWS_PER_TILE, _row, 0)


_sc_copy = pl.kernel(
    _sc_copy_body,
    out_type=jax.ShapeDtypeStruct((_LQ, _LK, _D), jnp.float32),
    mesh=plsc.VectorSubcoreMesh(
        core_axis_name="c", subcore_axis_name="s",
        num_cores=_NC, num_subcores=_NS,
    ),
    scratch_types=[
        pltpu.VMEM((_CH, _D), jnp.float32),
        pltpu.VMEM((_CH, _D), jnp.float32),
        pltpu.SemaphoreType.DMA,
        pltpu.SemaphoreType.DMA,
        pltpu.SemaphoreType.DMA,
    ],
)


def kernel(length_q, length_k, pe):
    off = jnp.asarray(length_k - length_q, jnp.int32).reshape((1,))
    t8 = _build_template(off, pe)
    return _sc_copy(t8)


# speedup vs baseline: 39.5896x; 1.0284x over previous
import jax
import jax.numpy as jnp
from jax import lax
from jax.experimental import pallas as pl
from jax.experimental.pallas import tpu as pltpu
from jax.experimental.pallas import tpu_sc as plsc

_LQ = 1024
_LK = 1024
_D = 256
_ROWS = 9
_MAXK = 4
_T = 4096
_MID = _T // 2 - 2

_NC = 2
_NS = 16
_ROWS_PER_TILE = _LQ // (_NC * _NS)


def _build_body(off_ref, pe_ref, t8_ref):
    s = pl.program_id(0)
    u = jax.lax.broadcasted_iota(jnp.int32, (_T, _D), 0)
    c = jnp.clip(u + s - _MID + off_ref[0], -_MAXK, _MAXK) + _MAXK
    acc = jnp.zeros((_T, _D), jnp.float32)
    for r in range(_ROWS):
        acc = jnp.where(c == r, pe_ref[r, :][None, :], acc)
    t8_ref[0] = acc


def _build_template(off, pe):
    return pl.pallas_call(
        _build_body,
        grid=(8,),
        in_specs=[
            pl.BlockSpec(memory_space=pltpu.SMEM),
            pl.BlockSpec((_ROWS, _D), lambda s: (0, 0)),
        ],
        out_specs=pl.BlockSpec((1, _T, _D), lambda s: (s, 0, 0)),
        out_shape=jax.ShapeDtypeStruct((8, _T, _D), jnp.float32),
    )(off, pe)


_CH = 32
_NCH = _LK // _CH
_RING = 8
_PF = 4
_M = _ROWS_PER_TILE * _NCH


def _sc_copy_body(t8_hbm, out_hbm, bufs, rsems, wsems):
    cid = lax.axis_index("c")
    sid = lax.axis_index("s")
    wid = cid * _NS + sid

    def _src(m):
        j = lax.shift_right_logical(m, 5)
        c = lax.bitwise_and(m, _NCH - 1)
        q = wid * _ROWS_PER_TILE + j
        start = _MID - q
        s = lax.rem(start, 8)
        base = pl.multiple_of(start - s, 8)
        return t8_hbm.at[s, pl.ds(base + _CH * c, _CH), :]

    def _dst(m):
        j = lax.shift_right_logical(m, 5)
        c = lax.bitwise_and(m, _NCH - 1)
        q = wid * _ROWS_PER_TILE + j
        return out_hbm.at[q, pl.ds(_CH * c, _CH), :]

    for u in range(_PF):
        pltpu.async_copy(_src(u), bufs[u], rsems[u])

    def _step(i, carry):
        for u in range(_RING):
            m = _RING * i + u
            pltpu.make_async_copy(_src(m), bufs[u], rsems[u]).wait()
            pltpu.async_copy(bufs[u], _dst(m), wsems[u])
            v = (u + _PF) % _RING
            mp = m + _PF

            def _prefetch(mp=mp, v=v):
                pltpu.make_async_copy(bufs[v], _dst(mp - _RING), wsems[v]).wait()
                pltpu.async_copy(_src(mp), bufs[v], rsems[v])

            if u < _PF:
                @pl.when(i == 0)
                def _(mp=mp, v=v):
                    pltpu.async_copy(_src(mp), bufs[v], rsems[v])

                @pl.when(i > 0)
                def _(pf=_prefetch):
                    pf()
            else:
                @pl.when(mp < _M)
                def _(pf=_prefetch):
                    pf()
        return carry

    lax.fori_loop(0, _M // _RING, _step, 0)

    for u in range(_PF):
        m = _M - _PF + u
        slot = m % _RING
        pltpu.make_async_copy(bufs[slot], _dst(m), wsems[slot]).wait()


_sc_copy = pl.kernel(
    _sc_copy_body,
    out_type=jax.ShapeDtypeStruct((_LQ, _LK, _D), jnp.float32),
    mesh=plsc.VectorSubcoreMesh(
        core_axis_name="c", subcore_axis_name="s",
        num_cores=_NC, num_subcores=_NS,
    ),
    scratch_types=[
        tuple(pltpu.VMEM((_CH, _D), jnp.float32) for _ in range(_RING)),
        tuple(pltpu.SemaphoreType.DMA for _ in range(_RING)),
        tuple(pltpu.SemaphoreType.DMA for _ in range(_RING)),
    ],
)


def kernel(length_q, length_k, pe):
    off = jnp.asarray(length_k - length_q, jnp.int32).reshape((1,))
    t8 = _build_template(off, pe)
    return _sc_copy(t8)


# speedup vs baseline: 50.6274x; 1.2788x over previous
import jax
import jax.numpy as jnp
from jax import lax
from jax.experimental import pallas as pl
from jax.experimental.pallas import tpu as pltpu
from jax.experimental.pallas import tpu_sc as plsc

_LQ = 1024
_LK = 1024
_D = 256
_ROWS = 9
_MAXK = 4
_T = 4096
_MID = _T // 2 - 2

_NC = 2
_NS = 16
_ROWS_PER_TILE = _LQ // (_NC * _NS)


def _build_body(off_ref, pe_ref, t8_ref):
    s = pl.program_id(0)
    u = jax.lax.broadcasted_iota(jnp.int32, (_T, _D), 0)
    c = jnp.clip(u + s - _MID + off_ref[0], -_MAXK, _MAXK) + _MAXK
    acc = jnp.zeros((_T, _D), jnp.float32)
    for r in range(_ROWS):
        acc = jnp.where(c == r, pe_ref[r, :][None, :], acc)
    t8_ref[0] = acc


def _build_template(off, pe):
    return pl.pallas_call(
        _build_body,
        grid=(8,),
        in_specs=[
            pl.BlockSpec(memory_space=pltpu.SMEM),
            pl.BlockSpec((_ROWS, _D), lambda s: (0, 0)),
        ],
        out_specs=pl.BlockSpec((1, _T, _D), lambda s: (s, 0, 0)),
        out_shape=jax.ShapeDtypeStruct((8, _T, _D), jnp.float32),
    )(off, pe)


_CK = 256
_NP = _LK // _CK
_WIN = 784


def _sc_copy_body(t8_hbm, out_hbm, shared, stage_sem, sem):
    cid = lax.axis_index("c")
    sid = lax.axis_index("s")
    wid = cid * _NS + sid
    b0 = pl.multiple_of(1528 - 512 * cid, 8)

    for kp in range(_NP):
        @pl.when(sid < 8)
        def _(kp=kp):
            pltpu.async_copy(
                t8_hbm.at[sid, pl.ds(pl.multiple_of(b0 + _CK * kp, 8), _WIN), :],
                shared.at[sid],
                stage_sem,
            ).wait()

        plsc.subcore_barrier()

        def _row(j, carry):
            q = wid * _ROWS_PER_TILE + j
            start = _MID - q
            s = lax.rem(start, 8)
            roff = pl.multiple_of(start - s - b0, 8)
            pltpu.async_copy(
                shared.at[s, pl.ds(roff, _CK), :],
                out_hbm.at[q, pl.ds(_CK * kp, _CK), :],
                sem,
            )
            return carry

        lax.fori_loop(0, _ROWS_PER_TILE, _row, 0)

        def _drain(j, carry):
            pltpu.make_async_copy(
                shared.at[0, pl.ds(0, _CK), :],
                out_hbm.at[0, pl.ds(0, _CK), :],
                sem,
            ).wait()
            return carry

        lax.fori_loop(0, _ROWS_PER_TILE, _drain, 0)

        plsc.subcore_barrier()


_sc_copy = pl.kernel(
    _sc_copy_body,
    out_type=jax.ShapeDtypeStruct((_LQ, _LK, _D), jnp.float32),
    mesh=plsc.VectorSubcoreMesh(
        core_axis_name="c", subcore_axis_name="s",
        num_cores=_NC, num_subcores=_NS,
    ),
    scratch_types=[
        pltpu.VMEM_SHARED((8, _WIN, _D), jnp.float32),
        pltpu.SemaphoreType.DMA,
        pltpu.SemaphoreType.DMA,
    ],
)


def kernel(length_q, length_k, pe):
    off = jnp.asarray(length_k - length_q, jnp.int32).reshape((1,))
    t8 = _build_template(off, pe)
    return _sc_copy(t8)
